# baseline (speedup 1.0000x reference)
import functools

import jax
import jax.numpy as jnp
from jax import lax
from jax.experimental import pallas as pl
from jax.experimental.pallas import tpu as pltpu
from jax.experimental.pallas import tpu_sc as plsc

_ETA = 10
_K = 32
_NC = 2
_NS = 16
_NW = _NC * _NS
_L = 16

_N_ENT = 1000000
_ALIGNED = 999936
_QCOLS = _ALIGNED // 4
_WC = 3968
_NCHUNK = _QCOLS // _WC
_TAIL = _N_ENT - _ALIGNED


@functools.lru_cache(maxsize=None)
def _build_detile():
    mesh = plsc.VectorSubcoreMesh(core_axis_name="c", subcore_axis_name="s")

    @functools.partial(
        pl.kernel,
        out_type=jax.ShapeDtypeStruct((2 * _K, _N_ENT), jnp.float32),
        mesh=mesh,
        compiler_params=pltpu.CompilerParams(
            needs_layout_passes=False, use_tc_tiling_on_sc=True),
        scratch_types=[
            pltpu.VMEM((8, _WC), jnp.float32),
            pltpu.VMEM((8, _WC), jnp.float32),
            pltpu.VMEM((_TAIL,), jnp.float32),
            pltpu.SemaphoreType.DMA,
            pltpu.SemaphoreType.DMA,
        ],
    )
    def detile(entT_hbm, relT_hbm, tail_hbm, out_hbm,
               buf0_v, buf1_v, tail_v, sem0, sem1):
        cid = lax.axis_index("c")
        sid = lax.axis_index("s")
        wid = sid * _NC + cid
        rg = wid % 16
        kr = rg // 4
        q = rg % 4
        row0 = pl.multiple_of(8 * kr, 8)
        col0 = pl.multiple_of(q * _QCOLS, 128)

        bufs = (buf0_v, buf1_v)
        sems = (sem0, sem1)

        def run(tab_hbm, out_row0):
            cps = [None, None]
            cps[0] = pltpu.async_copy(
                tab_hbm.at[pl.ds(row0, 8), pl.ds(col0, _WC)], buf0_v, sem0)
            for j in range(_NCHUNK):
                cps[j % 2].wait()
                if j + 1 < _NCHUNK:
                    c_next = pl.multiple_of(col0 + (j + 1) * _WC, 128)
                    cps[(j + 1) % 2] = pltpu.async_copy(
                        tab_hbm.at[pl.ds(row0, 8), pl.ds(c_next, _WC)],
                        bufs[(j + 1) % 2], sems[(j + 1) % 2])
                pltpu.sync_copy(
                    bufs[j % 2],
                    out_hbm.at[pl.ds(out_row0 + row0, 8),
                               pl.ds(col0 + j * _WC, _WC)])

        @pl.when(wid < 16)
        def _():
            run(entT_hbm, 0)

        @pl.when(wid >= 16)
        def _():
            run(relT_hbm, _K)

        for i in range(2):
            r = wid * 2 + i
            pltpu.sync_copy(tail_hbm.at[pl.ds(r * _TAIL, _TAIL)], tail_v)
            pltpu.sync_copy(tail_v, out_hbm.at[r, pl.ds(_ALIGNED, _TAIL)])

    return detile


@functools.lru_cache(maxsize=None)
def _build_score(n: int):
    C = n // _NW
    V = C // _L
    mesh = plsc.VectorSubcoreMesh(core_axis_name="c", subcore_axis_name="s")

    @functools.partial(
        pl.kernel,
        out_type=(
            jax.ShapeDtypeStruct((n,), jnp.float32),
            jax.ShapeDtypeStruct((n * _ETA,), jnp.float32),
        ),
        mesh=mesh,
        compiler_params=pltpu.CompilerParams(
            needs_layout_passes=False, use_tc_tiling_on_sc=False),
        scratch_types=[
            pltpu.VMEM((C,), jnp.int32),
            pltpu.VMEM((C,), jnp.int32),
            pltpu.VMEM((C,), jnp.int32),
            pltpu.VMEM((_ETA * C,), jnp.int32),
            pltpu.VMEM((_ETA, C), jnp.int32),
            pltpu.VMEM((2 * C,), jnp.float32),
            pltpu.VMEM((C,), jnp.float32),
            pltpu.VMEM((_ETA * C,), jnp.float32),
            pltpu.VMEM((C,), jnp.float32),
            pltpu.VMEM((_ETA, C), jnp.float32),
            pltpu.SemaphoreType.DMA,
            pltpu.SemaphoreType.DMA,
            pltpu.SemaphoreType.DMA,
            pltpu.SemaphoreType.DMA,
        ],
    )
    def score(tabL_hbm, s_hbm, p_hbm, o_hbm, repl_hbm, sel_hbm,
              out_inp, out_corr,
              s_v, p_v, o_v, repl_v, sel_v, eseo_v, ep_v, er_v,
              acc_inp, acc_corr, sem0, sem1, sem2, sem3):
        cid = lax.axis_index("c")
        sid = lax.axis_index("s")
        wid = sid * _NC + cid
        base_row = wid * C

        pltpu.sync_copy(s_hbm.at[wid], s_v)
        pltpu.sync_copy(p_hbm.at[wid], p_v)
        pltpu.sync_copy(o_hbm.at[wid], o_v)
        pltpu.sync_copy(repl_hbm.at[wid], repl_v)
        pltpu.sync_copy(sel_hbm.at[wid], sel_v)

        zeros = jnp.zeros((_L,), jnp.float32)
        for i in range(V):
            acc_inp[pl.ds(i * _L, _L)] = zeros
            for t in range(_ETA):
                acc_corr[t, pl.ds(i * _L, _L)] = zeros

        def body(k, carry):
            cp_es = pltpu.async_copy(
                tabL_hbm.at[k].at[s_v], eseo_v.at[pl.ds(0, C)], sem0)
            cp_eo = pltpu.async_copy(
                tabL_hbm.at[k].at[o_v], eseo_v.at[pl.ds(C, C)], sem1)
            cp_ep = pltpu.async_copy(
                tabL_hbm.at[_K + k].at[p_v], ep_v, sem2)
            cp_er = pltpu.async_copy(
                tabL_hbm.at[k].at[repl_v], er_v, sem3)
            cp_es.wait()
            cp_eo.wait()
            cp_ep.wait()
            cp_er.wait()

            for i in range(V):
                b = i * _L
                es = eseo_v[pl.ds(b, _L)]
                eo = eseo_v[pl.ds(C + b, _L)]
                ep = ep_v[pl.ds(b, _L)]
                acc_inp[pl.ds(b, _L)] += es * ep * eo
                for t in range(_ETA):
                    sel16 = sel_v[t, pl.ds(b, _L)]
                    cv = plsc.load_gather(eseo_v, [sel16])
                    rv = er_v[pl.ds(t * C + b, _L)]
                    acc_corr[t, pl.ds(b, _L)] += cv * ep * rv
            return carry

        lax.fori_loop(0, _K, body, 0)

        pltpu.sync_copy(acc_inp, out_inp.at[pl.ds(base_row, C)])
        for t in range(_ETA):
            pltpu.sync_copy(acc_corr.at[t],
                            out_corr.at[pl.ds(t * n + base_row, C)])

    return score


def kernel(inputs, ent_emb, rel_emb):
    n = inputs.shape[0]
    n_ent = ent_emb.shape[0]
    C = n // _NW

    km, kr = jax.random.split(jax.random.key(42))
    keep_subj = jax.random.randint(km, (n * _ETA,), 0, 2, dtype=jnp.int32)
    replacements = jax.random.randint(kr, (n * _ETA,), 0, n_ent,
                                      dtype=jnp.int32)
    keep_obj = 1 - keep_subj

    s = inputs[:, 0].reshape(_NW, C)
    p = inputs[:, 1].reshape(_NW, C)
    o = inputs[:, 2].reshape(_NW, C)
    repl = replacements.reshape(_ETA, _NW, C).transpose(1, 0, 2)
    repl_flat = repl.reshape(_NW, _ETA * C)
    sel = (jnp.arange(C, dtype=jnp.int32)[None, None, :]
           + C * keep_obj.reshape(_ETA, _NW, C).transpose(1, 0, 2))

    tail = jnp.concatenate(
        [ent_emb[_ALIGNED:, :].T.reshape(-1),
         rel_emb[_ALIGNED:, :].T.reshape(-1)])

    tabL = _build_detile()(ent_emb.T, rel_emb.T, tail)
    inp_score, corr_score = _build_score(n)(
        tabL, s, p, o, repl_flat, sel)
    return (inp_score, corr_score)
